# Initial kernel scaffold; baseline (speedup 1.0000x reference)
import jax
import jax.numpy as jnp
from jax.experimental import pallas as pl


def kernel(batch_imgs, batch_labels, batch_titles, W):
    raise NotImplementedError("write your pallas kernel here")


# speedup vs baseline: 12.8885x; 12.8885x over previous
import functools
import numpy as np
import jax
import jax.numpy as jnp
from jax.experimental import pallas as pl

ALPHA = 0.2
IMAX = np.int32(0x7FFFFFFF)


def _embs_body(x_ref, w_ref, embs_ref, sq_ref):
    e = jnp.dot(x_ref[...], w_ref[...], preferred_element_type=jnp.float32)
    embs_ref[...] = e
    sq_ref[...] = jnp.sum(e * e, axis=1, keepdims=True)


def _ce_roll(key, iota0, k, j):
    up = jnp.roll(key, -j, axis=0)
    down = jnp.roll(key, j, axis=0)
    upper = (iota0 & j) != 0
    pv = jnp.where(upper, down, up)
    asc = (iota0 & k) == 0
    mn = jnp.minimum(key, pv)
    mx = jnp.maximum(key, pv)
    take_min = asc ^ upper
    return jnp.where(take_min, mn, mx)


def _mine_body(embs_ref, sqc_ref, sqr_ref, labc_ref, labr_ref,
               loss_ref, cnt_ref, *, blk, n):
    b = pl.program_id(0)
    embs = embs_ref[...]
    embs_a = embs_ref[pl.ds(b * blk, blk), :]
    g = jax.lax.dot_general(embs, embs_a, (((1,), (1,)), ((), ())),
                            preferred_element_type=jnp.float32)
    d = sqc_ref[...] + sqr_ref[:, pl.ds(b * blk, blk)] - 2.0 * g
    d = jnp.maximum(d, 0.0)

    labc = labc_ref[...]
    labr = labr_ref[:, pl.ds(b * blk, blk)]
    same = labc == labr
    jidx = jax.lax.broadcasted_iota(jnp.int32, (n, blk), 0)
    aidx = jax.lax.broadcasted_iota(jnp.int32, (n, blk), 1) + b * blk
    pos = same & (aidx < jidx)
    tag = jnp.where(same, jnp.where(pos, 1, 2), 0)

    firstneg = jnp.min(jnp.where(same, n, jidx), axis=0, keepdims=True)
    firstneg = jnp.where(firstneg == n, 0, firstneg)
    fallback = jnp.sum(jnp.where(jidx == firstneg, d, 0.0),
                       axis=0, keepdims=True)

    kbits = jax.lax.bitcast_convert_type(d, jnp.int32)
    key = (kbits & ~np.int32(3)) | tag

    iota0 = jax.lax.broadcasted_iota(jnp.int32, (n, 1), 0)
    k = 2
    while k <= n:
        j = k // 2
        while j >= 1:
            key = _ce_roll(key, iota0, k, j)
            j //= 2
        k *= 2

    m = jnp.where((key & 3) == 0, key, IMAX)
    s = 1
    while s < n:
        shifted = jnp.concatenate(
            [m[s:], jnp.full((s, blk), IMAX, jnp.int32)], axis=0)
        m = jnp.minimum(m, shifted)
        s *= 2

    has = m != IMAX
    candv = jax.lax.bitcast_convert_type(m, jnp.float32)
    val = jax.lax.bitcast_convert_type(key & ~np.int32(3), jnp.float32)
    dneg = jnp.where(has, candv, fallback)
    terms = jnp.where((key & 3) == 1,
                      jnp.maximum(val - dneg + ALPHA, 0.0), 0.0)
    part = jnp.sum(terms).reshape(1, 1)
    cnt = jnp.sum(((key & 3) == 1).astype(jnp.float32)).reshape(1, 1)

    @pl.when(b == 0)
    def _():
        loss_ref[...] = jnp.zeros((1, 1), jnp.float32)
        cnt_ref[...] = jnp.zeros((1, 1), jnp.float32)

    loss_ref[...] += part
    cnt_ref[...] += cnt


def kernel(batch_imgs, batch_labels, batch_titles, W):
    n, d_in = batch_imgs.shape
    d_emb = W.shape[1]
    assert (n & (n - 1)) == 0, "batch size must be a power of two"
    blk = min(128, n)
    nblk = n // blk

    embs, sqc = pl.pallas_call(
        _embs_body,
        out_shape=[jax.ShapeDtypeStruct((n, d_emb), jnp.float32),
                   jax.ShapeDtypeStruct((n, 1), jnp.float32)],
    )(batch_imgs, W)

    sqr = sqc.reshape(1, n)
    labc = batch_labels.reshape(n, 1).astype(jnp.int32)
    labr = batch_labels.reshape(1, n).astype(jnp.int32)

    loss_sum, cnt = pl.pallas_call(
        functools.partial(_mine_body, blk=blk, n=n),
        grid=(nblk,),
        in_specs=[
            pl.BlockSpec((n, d_emb), lambda b: (0, 0)),
            pl.BlockSpec((n, 1), lambda b: (0, 0)),
            pl.BlockSpec((1, n), lambda b: (0, 0)),
            pl.BlockSpec((n, 1), lambda b: (0, 0)),
            pl.BlockSpec((1, n), lambda b: (0, 0)),
        ],
        out_specs=[
            pl.BlockSpec((1, 1), lambda b: (0, 0)),
            pl.BlockSpec((1, 1), lambda b: (0, 0)),
        ],
        out_shape=[jax.ShapeDtypeStruct((1, 1), jnp.float32),
                   jax.ShapeDtypeStruct((1, 1), jnp.float32)],
    )(embs, sqc, sqr, labc, labr)

    return loss_sum[0, 0] / jnp.maximum(cnt[0, 0], 1.0)
